# baseline (speedup 1.0000x reference)
import functools

import jax
import jax.numpy as jnp
from jax import lax
from jax.experimental import pallas as pl
from jax.experimental.pallas import tpu as pltpu
from jax.experimental.pallas import tpu_sc as plsc

B, C, H, W = 8, 96, 32, 32
K = 9
N = H * W
C2 = 2 * C
BN_EPS = 1e-5
BN_TOT = B * N

CP = 256
NW = 32
NPW = BN_TOT // NW
CH = 8
NCH = NPW // CH

_NEG = -3.0e38


def _tc1_body(x_ref, w1e_ref, b1e_ref, wat_ref, w2t_ref, gcb_ref,
              a_ref, bv_ref, idx_ref):
    b = pl.program_id(0)
    xb = x_ref[0]
    h = jnp.dot(w1e_ref[...], xb,
                preferred_element_type=jnp.float32) + b1e_ref[...]
    nrm2 = jnp.sum(h * h, axis=0, keepdims=True)
    denom = jnp.maximum(jnp.sqrt(nrm2), 1e-12)
    xn = h / denom
    d = lax.dot_general(xn, xn, (((0,), (0,)), ((), ())),
                        preferred_element_type=jnp.float32)
    sq = jnp.sum(xn * xn, axis=0, keepdims=True)
    cur = 2.0 * d - sq
    iota = lax.broadcasted_iota(jnp.int32, (N, N), 1)
    cols = []
    vp = None
    for t in range(K):
        if t == 0:
            masked = cur
        else:
            live = (cur < vp) | ((cur == vp) & (iota > jp))
            masked = jnp.where(live, cur, _NEG)
        vp = jnp.max(masked, axis=1, keepdims=True)
        jp = jnp.min(jnp.where(masked >= vp, iota, N), axis=1, keepdims=True)
        cols.append(jp)
    idx_ref[0] = jnp.concatenate(cols, axis=1) + b * N
    a_ref[0] = lax.dot_general(h, wat_ref[...], (((0,), (0,)), ((), ())),
                               preferred_element_type=jnp.float32) + gcb_ref[...]
    bvv = lax.dot_general(h, w2t_ref[...], (((0,), (0,)), ((), ())),
                          preferred_element_type=jnp.float32)
    bv_ref[0, :, :C2] = bvv.astype(jnp.bfloat16)


def _tc1_call(x3, w1e, b1e, wat, w2t, gcb, interpret=False):
    return pl.pallas_call(
        _tc1_body,
        grid=(B,),
        in_specs=[
            pl.BlockSpec((1, C, N), lambda b: (b, 0, 0)),
            pl.BlockSpec((C, C), lambda b: (0, 0)),
            pl.BlockSpec((C, 1), lambda b: (0, 0)),
            pl.BlockSpec((C, C2), lambda b: (0, 0)),
            pl.BlockSpec((C, C2), lambda b: (0, 0)),
            pl.BlockSpec((1, C2), lambda b: (0, 0)),
        ],
        out_specs=[
            pl.BlockSpec((1, N, C2), lambda b: (b, 0, 0)),
            pl.BlockSpec((1, N, CP), lambda b: (b, 0, 0)),
            pl.BlockSpec((1, N, K), lambda b: (b, 0, 0)),
        ],
        out_shape=[
            jax.ShapeDtypeStruct((B, N, C2), jnp.float32),
            jax.ShapeDtypeStruct((B, N, CP), jnp.bfloat16),
            jax.ShapeDtypeStruct((B, N, K), jnp.int32),
        ],
        interpret=interpret,
    )(x3, w1e, b1e, wat, w2t, gcb)


def _sc_body(bv_hbm, idx_hbm, out_hbm, idx_v,
             rows0, rows1, m0, m1, semg0, semg1, semo0, semo1):
    wid = lax.axis_index("s") * 2 + lax.axis_index("c")
    base = wid * NPW
    pltpu.sync_copy(idx_hbm.at[wid], idx_v)
    rows = (rows0, rows1)
    mv = (m0, m1)
    semg = (semg0, semg1)
    semo = (semo0, semo1)
    pltpu.async_copy(bv_hbm.at[idx_v.at[0]], rows0, semg0)
    pltpu.async_copy(bv_hbm.at[idx_v.at[1]], rows1, semg1)

    def pair_body(p, carry):
        for par in range(2):
            ch = p * 2 + par
            pltpu.make_async_copy(bv_hbm.at[idx_v.at[ch]],
                                  rows[par], semg[par]).wait()

            @pl.when(p > 0)
            def _wait_out():
                pltpu.make_async_copy(mv[par],
                                      out_hbm.at[pl.ds(0, CH)],
                                      semo[par]).wait()

            himask = jnp.int32(-65536)
            lomask = jnp.int32(65535)
            for i in range(CH):
                for c in range(C2 // 32):
                    sl = pl.ds(c * 16, 16)
                    me = mo = None
                    for k2 in range(K):
                        v = rows[par][i * K + k2, sl]
                        fe = plsc.bitcast(jnp.left_shift(v, 16), jnp.float32)
                        fo = plsc.bitcast(v & himask, jnp.float32)
                        me = fe if me is None else jnp.maximum(me, fe)
                        mo = fo if mo is None else jnp.maximum(mo, fo)
                    ie = jnp.right_shift(plsc.bitcast(me, jnp.int32), 16)
                    io = plsc.bitcast(mo, jnp.int32) & himask
                    mv[par][i, sl] = (ie & lomask) | io
            pltpu.async_copy(mv[par], out_hbm.at[pl.ds(base + ch * CH, CH)],
                             semo[par])

            @pl.when(ch + 2 < NCH)
            def _next_gather():
                pltpu.async_copy(bv_hbm.at[idx_v.at[ch + 2]], rows[par],
                                 semg[par])
        return carry
    lax.fori_loop(0, NCH // 2, pair_body, 0)
    for par in range(2):
        pltpu.make_async_copy(mv[par], out_hbm.at[pl.ds(0, CH)],
                              semo[par]).wait()


def _sc_call(bv_flat, idx_3d, interpret=False):
    mesh = plsc.VectorSubcoreMesh(core_axis_name="c", subcore_axis_name="s")
    fn = functools.partial(
        pl.kernel,
        out_type=jax.ShapeDtypeStruct((BN_TOT, C2 // 2), jnp.int32),
        mesh=mesh,
        scratch_types=[
            pltpu.VMEM((NCH, CH * K), jnp.int32),
            pltpu.VMEM((CH * K, CP // 2), jnp.int32),
            pltpu.VMEM((CH * K, CP // 2), jnp.int32),
            pltpu.VMEM((CH, C2 // 2), jnp.int32),
            pltpu.VMEM((CH, C2 // 2), jnp.int32),
            pltpu.SemaphoreType.DMA,
            pltpu.SemaphoreType.DMA,
            pltpu.SemaphoreType.DMA,
            pltpu.SemaphoreType.DMA,
        ],
        compiler_params=pltpu.CompilerParams(needs_layout_passes=False),
        interpret=interpret,
    )(_sc_body)
    return fn(bv_flat, idx_3d)


def _tc2_body(a_ref, m_ref, w2e_ref, b2e_ref, x_ref, o_ref):
    g = jnp.maximum(a_ref[0] + m_ref[0].astype(jnp.float32), 0.0)
    o = lax.dot_general(w2e_ref[...], g, (((1,), (1,)), ((), ())),
                        preferred_element_type=jnp.float32)
    o_ref[0] = o + b2e_ref[...] + x_ref[0]


def _tc2_call(apre, m, w2e, b2e, x3, interpret=False):
    return pl.pallas_call(
        _tc2_body,
        grid=(B,),
        in_specs=[
            pl.BlockSpec((1, N, C2), lambda b: (b, 0, 0)),
            pl.BlockSpec((1, N, C2), lambda b: (b, 0, 0)),
            pl.BlockSpec((C, C2), lambda b: (0, 0)),
            pl.BlockSpec((C, 1), lambda b: (0, 0)),
            pl.BlockSpec((1, C, N), lambda b: (b, 0, 0)),
        ],
        out_specs=pl.BlockSpec((1, C, N), lambda b: (b, 0, 0)),
        out_shape=jax.ShapeDtypeStruct((B, C, N), jnp.float32),
        interpret=interpret,
    )(apre, m, w2e, b2e, x3)


def _run(x, fc1_w, fc1_b, bn1_g, bn1_b, gc_w, gc_b, fc2_w, fc2_b, bn2_g, bn2_b,
         interpret=False, sc_interpret=False):
    x3 = x.reshape(B, C, N)
    s1 = bn1_g * lax.rsqrt(jnp.float32(1.0 + BN_EPS))
    w1e = fc1_w * s1[:, None]
    b1e = (fc1_b * s1 + bn1_b)[:, None]
    w1 = gc_w[:, :C]
    w2 = gc_w[:, C:]
    wat = (w1 - w2).T
    w2t = w2.T
    gcb = gc_b[None, :]
    s2 = bn2_g * lax.rsqrt(jnp.float32(1.0 + BN_EPS))
    w2e = fc2_w * s2[:, None]
    b2e = (fc2_b * s2 + bn2_b)[:, None]

    apre, bv, idx = _tc1_call(x3, w1e, b1e, wat, w2t, gcb, interpret=interpret)
    bv_i32 = lax.bitcast_convert_type(bv.reshape(BN_TOT, CP // 2, 2), jnp.int32)
    idx_3d = idx.reshape(NW, NCH, CH * K)
    m = _sc_call(bv_i32, idx_3d, interpret=sc_interpret)
    m3 = lax.bitcast_convert_type(m, jnp.bfloat16).reshape(B, N, C2)
    out = _tc2_call(apre, m3, w2e, b2e, x3, interpret=interpret)
    return out.reshape(B, C, H, W)


def kernel(x, fc1_w, fc1_b, bn1_g, bn1_b, gc_w, gc_b, fc2_w, fc2_b, bn2_g, bn2_b):
    return _run(x, fc1_w, fc1_b, bn1_g, bn1_b, gc_w, gc_b,
                fc2_w, fc2_b, bn2_g, bn2_b)


# speedup vs baseline: 1.0381x; 1.0381x over previous
import functools

import jax
import jax.numpy as jnp
from jax import lax
from jax.experimental import pallas as pl
from jax.experimental.pallas import tpu as pltpu
from jax.experimental.pallas import tpu_sc as plsc

B, C, H, W = 8, 96, 32, 32
K = 9
N = H * W
C2 = 2 * C
BN_EPS = 1e-5
BN_TOT = B * N

CP = 256
NW = 32
NPW = BN_TOT // NW
CH = 8
NCH = NPW // CH

_NEG = -3.0e38


def _tc1_body(x_ref, w1e_ref, b1e_ref, wat_ref, w2t_ref, gcb_ref,
              a_ref, bv_ref, idx_ref):
    b = pl.program_id(0)
    xb = x_ref[0]
    h = jnp.dot(w1e_ref[...], xb,
                preferred_element_type=jnp.float32) + b1e_ref[...]
    nrm2 = jnp.sum(h * h, axis=0, keepdims=True)
    denom = jnp.maximum(jnp.sqrt(nrm2), 1e-12)
    xn = h / denom
    d = lax.dot_general(xn, xn, (((0,), (0,)), ((), ())),
                        preferred_element_type=jnp.float32)
    sq = jnp.sum(xn * xn, axis=0, keepdims=True)
    cur = 2.0 * d - sq
    iota = lax.broadcasted_iota(jnp.int32, (N, N), 1)
    cols = []
    vp = None
    for t in range(K):
        if t == 0:
            masked = cur
        else:
            live = (cur < vp) | ((cur == vp) & (iota > jp))
            masked = jnp.where(live, cur, _NEG)
        vp = jnp.max(masked, axis=1, keepdims=True)
        jp = jnp.min(jnp.where(masked >= vp, iota, N), axis=1, keepdims=True)
        cols.append(jp)
    idx_ref[0] = jnp.concatenate(cols, axis=1) + b * N
    a_ref[0] = lax.dot_general(h, wat_ref[...], (((0,), (0,)), ((), ())),
                               preferred_element_type=jnp.float32) + gcb_ref[...]
    bv_ref[0, :, :C2] = lax.dot_general(h, w2t_ref[...],
                                        (((0,), (0,)), ((), ())),
                                        preferred_element_type=jnp.float32)


def _tc1_call(x3, w1e, b1e, wat, w2t, gcb, interpret=False):
    return pl.pallas_call(
        _tc1_body,
        grid=(B,),
        in_specs=[
            pl.BlockSpec((1, C, N), lambda b: (b, 0, 0)),
            pl.BlockSpec((C, C), lambda b: (0, 0)),
            pl.BlockSpec((C, 1), lambda b: (0, 0)),
            pl.BlockSpec((C, C2), lambda b: (0, 0)),
            pl.BlockSpec((C, C2), lambda b: (0, 0)),
            pl.BlockSpec((1, C2), lambda b: (0, 0)),
        ],
        out_specs=[
            pl.BlockSpec((1, N, C2), lambda b: (b, 0, 0)),
            pl.BlockSpec((1, N, CP), lambda b: (b, 0, 0)),
            pl.BlockSpec((1, N, K), lambda b: (b, 0, 0)),
        ],
        out_shape=[
            jax.ShapeDtypeStruct((B, N, C2), jnp.float32),
            jax.ShapeDtypeStruct((B, N, CP), jnp.float32),
            jax.ShapeDtypeStruct((B, N, K), jnp.int32),
        ],
        interpret=interpret,
    )(x3, w1e, b1e, wat, w2t, gcb)


def _sc_body(bv_hbm, idx_hbm, out_hbm, idx_v,
             rows0, rows1, m0, m1, semg0, semg1, semo0, semo1):
    wid = lax.axis_index("s") * 2 + lax.axis_index("c")
    base = wid * NPW
    pltpu.sync_copy(idx_hbm.at[wid], idx_v)
    rows = (rows0, rows1)
    mv = (m0, m1)
    semg = (semg0, semg1)
    semo = (semo0, semo1)
    pltpu.async_copy(bv_hbm.at[idx_v.at[0]], rows0, semg0)
    pltpu.async_copy(bv_hbm.at[idx_v.at[1]], rows1, semg1)

    def pair_body(p, carry):
        for par in range(2):
            ch = p * 2 + par
            pltpu.make_async_copy(bv_hbm.at[idx_v.at[ch]],
                                  rows[par], semg[par]).wait()

            @pl.when(p > 0)
            def _wait_out():
                pltpu.make_async_copy(mv[par],
                                      out_hbm.at[pl.ds(0, CH)],
                                      semo[par]).wait()

            for i in range(CH):
                for c in range(C2 // 16):
                    sl = pl.ds(c * 16, 16)
                    m = rows[par][i * K, sl]
                    for k2 in range(1, K):
                        m = jnp.maximum(m, rows[par][i * K + k2, sl])
                    mv[par][i, sl] = m
            pltpu.async_copy(mv[par], out_hbm.at[pl.ds(base + ch * CH, CH)],
                             semo[par])

            @pl.when(ch + 2 < NCH)
            def _next_gather():
                pltpu.async_copy(bv_hbm.at[idx_v.at[ch + 2]], rows[par],
                                 semg[par])
        return carry
    lax.fori_loop(0, NCH // 2, pair_body, 0)
    for par in range(2):
        pltpu.make_async_copy(mv[par], out_hbm.at[pl.ds(0, CH)],
                              semo[par]).wait()


def _sc_call(bv_flat, idx_3d, interpret=False):
    mesh = plsc.VectorSubcoreMesh(core_axis_name="c", subcore_axis_name="s")
    fn = functools.partial(
        pl.kernel,
        out_type=jax.ShapeDtypeStruct((BN_TOT, C2), jnp.float32),
        mesh=mesh,
        scratch_types=[
            pltpu.VMEM((NCH, CH * K), jnp.int32),
            pltpu.VMEM((CH * K, CP), jnp.float32),
            pltpu.VMEM((CH * K, CP), jnp.float32),
            pltpu.VMEM((CH, C2), jnp.float32),
            pltpu.VMEM((CH, C2), jnp.float32),
            pltpu.SemaphoreType.DMA,
            pltpu.SemaphoreType.DMA,
            pltpu.SemaphoreType.DMA,
            pltpu.SemaphoreType.DMA,
        ],
        interpret=interpret,
    )(_sc_body)
    return fn(bv_flat, idx_3d)


def _tc2_body(a_ref, m_ref, w2e_ref, b2e_ref, x_ref, o_ref):
    g = jnp.maximum(a_ref[0] + m_ref[0], 0.0)
    o = lax.dot_general(w2e_ref[...], g, (((1,), (1,)), ((), ())),
                        preferred_element_type=jnp.float32)
    o_ref[0] = o + b2e_ref[...] + x_ref[0]


def _tc2_call(apre, m, w2e, b2e, x3, interpret=False):
    return pl.pallas_call(
        _tc2_body,
        grid=(B,),
        in_specs=[
            pl.BlockSpec((1, N, C2), lambda b: (b, 0, 0)),
            pl.BlockSpec((1, N, C2), lambda b: (b, 0, 0)),
            pl.BlockSpec((C, C2), lambda b: (0, 0)),
            pl.BlockSpec((C, 1), lambda b: (0, 0)),
            pl.BlockSpec((1, C, N), lambda b: (b, 0, 0)),
        ],
        out_specs=pl.BlockSpec((1, C, N), lambda b: (b, 0, 0)),
        out_shape=jax.ShapeDtypeStruct((B, C, N), jnp.float32),
        interpret=interpret,
    )(apre, m, w2e, b2e, x3)


def _run(x, fc1_w, fc1_b, bn1_g, bn1_b, gc_w, gc_b, fc2_w, fc2_b, bn2_g, bn2_b,
         interpret=False, sc_interpret=False):
    x3 = x.reshape(B, C, N)
    s1 = bn1_g * lax.rsqrt(jnp.float32(1.0 + BN_EPS))
    w1e = fc1_w * s1[:, None]
    b1e = (fc1_b * s1 + bn1_b)[:, None]
    w1 = gc_w[:, :C]
    w2 = gc_w[:, C:]
    wat = (w1 - w2).T
    w2t = w2.T
    gcb = gc_b[None, :]
    s2 = bn2_g * lax.rsqrt(jnp.float32(1.0 + BN_EPS))
    w2e = fc2_w * s2[:, None]
    b2e = (fc2_b * s2 + bn2_b)[:, None]

    apre, bv, idx = _tc1_call(x3, w1e, b1e, wat, w2t, gcb, interpret=interpret)
    bv_flat = bv.reshape(BN_TOT, CP)
    idx_3d = idx.reshape(NW, NCH, CH * K)
    m = _sc_call(bv_flat, idx_3d, interpret=sc_interpret)
    m3 = m.reshape(B, N, C2)
    out = _tc2_call(apre, m3, w2e, b2e, x3, interpret=interpret)
    return out.reshape(B, C, H, W)


def kernel(x, fc1_w, fc1_b, bn1_g, bn1_b, gc_w, gc_b, fc2_w, fc2_b, bn2_g, bn2_b):
    return _run(x, fc1_w, fc1_b, bn1_g, bn1_b, gc_w, gc_b,
                fc2_w, fc2_b, bn2_g, bn2_b)


# speedup vs baseline: 1.3436x; 1.2943x over previous
import functools

import jax
import jax.numpy as jnp
from jax import lax
from jax.experimental import pallas as pl
from jax.experimental.pallas import tpu as pltpu
from jax.experimental.pallas import tpu_sc as plsc

B, C, H, W = 8, 96, 32, 32
K = 9
N = H * W
C2 = 2 * C
BN_EPS = 1e-5
BN_TOT = B * N

CP = 256
NW = 32
NPW = BN_TOT // NW
CH = 8
NCH = NPW // CH

_NEG = -3.0e38


def _tc1_body(x_ref, w1e_ref, b1e_ref, wat_ref, w2t_ref, gcb_ref,
              a_ref, bv_ref, idx_ref):
    b = pl.program_id(0)
    xb = x_ref[0]
    h = jnp.dot(w1e_ref[...], xb,
                preferred_element_type=jnp.float32) + b1e_ref[...]
    nrm2 = jnp.sum(h * h, axis=0, keepdims=True)
    denom = jnp.maximum(jnp.sqrt(nrm2), 1e-12)
    xn = h / denom
    d = lax.dot_general(xn, xn, (((0,), (0,)), ((), ())),
                        preferred_element_type=jnp.float32)
    sq = jnp.sum(xn * xn, axis=0, keepdims=True)
    cur = 2.0 * d - sq
    iota = lax.broadcasted_iota(jnp.int32, (N, N), 1)
    cols = []
    vp = None
    for t in range(K):
        if t == 0:
            masked = cur
        else:
            live = (cur < vp) | ((cur == vp) & (iota > jp))
            masked = jnp.where(live, cur, _NEG)
        vp = jnp.max(masked, axis=1, keepdims=True)
        jp = jnp.min(jnp.where(masked >= vp, iota, N), axis=1, keepdims=True)
        cols.append(jp)
    idx_ref[0] = jnp.concatenate(cols, axis=1) + b * N
    a_ref[0] = lax.dot_general(h, wat_ref[...], (((0,), (0,)), ((), ())),
                               preferred_element_type=jnp.float32) + gcb_ref[...]
    bv_ref[0, :, :C2] = lax.dot_general(h, w2t_ref[...],
                                        (((0,), (0,)), ((), ())),
                                        preferred_element_type=jnp.float32)


def _tc1_call(x3, w1e, b1e, wat, w2t, gcb, interpret=False):
    return pl.pallas_call(
        _tc1_body,
        grid=(B,),
        in_specs=[
            pl.BlockSpec((1, C, N), lambda b: (b, 0, 0)),
            pl.BlockSpec((C, C), lambda b: (0, 0)),
            pl.BlockSpec((C, 1), lambda b: (0, 0)),
            pl.BlockSpec((C, C2), lambda b: (0, 0)),
            pl.BlockSpec((C, C2), lambda b: (0, 0)),
            pl.BlockSpec((1, C2), lambda b: (0, 0)),
        ],
        out_specs=[
            pl.BlockSpec((1, N, C2), lambda b: (b, 0, 0)),
            pl.BlockSpec((1, N, CP), lambda b: (b, 0, 0)),
            pl.BlockSpec((1, N, K), lambda b: (b, 0, 0)),
        ],
        out_shape=[
            jax.ShapeDtypeStruct((B, N, C2), jnp.float32),
            jax.ShapeDtypeStruct((B, N, CP), jnp.float32),
            jax.ShapeDtypeStruct((B, N, K), jnp.int32),
        ],
        interpret=interpret,
    )(x3, w1e, b1e, wat, w2t, gcb)


def _sc_body(bv_hbm, idx_hbm, out_hbm, idx_v,
             rows0, rows1, m0, m1, semg0, semg1, semo0, semo1):
    wid = lax.axis_index("s") * 2 + lax.axis_index("c")
    base = wid * NPW
    pltpu.sync_copy(idx_hbm.at[wid], idx_v)
    rows = (rows0, rows1)
    mv = (m0, m1)
    semg = (semg0, semg1)
    semo = (semo0, semo1)
    pltpu.async_copy(bv_hbm.at[idx_v.at[0]], rows0, semg0)
    pltpu.async_copy(bv_hbm.at[idx_v.at[1]], rows1, semg1)

    def pair_body(p, carry):
        for par in range(2):
            ch = p * 2 + par
            pltpu.make_async_copy(bv_hbm.at[idx_v.at[ch]],
                                  rows[par], semg[par]).wait()

            @pl.when(p > 0)
            def _wait_out():
                pltpu.make_async_copy(mv[par],
                                      out_hbm.at[pl.ds(0, CH)],
                                      semo[par]).wait()

            def node_body(i, c2):
                for c in range(C2 // 16):
                    sl = pl.ds(c * 16, 16)
                    m = rows[par][i * K, sl]
                    for k2 in range(1, K):
                        m = jnp.maximum(m, rows[par][i * K + k2, sl])
                    mv[par][i, sl] = m
                return c2
            lax.fori_loop(0, CH, node_body, 0)
            pltpu.async_copy(mv[par], out_hbm.at[pl.ds(base + ch * CH, CH)],
                             semo[par])

            @pl.when(ch + 2 < NCH)
            def _next_gather():
                pltpu.async_copy(bv_hbm.at[idx_v.at[ch + 2]], rows[par],
                                 semg[par])
        return carry
    lax.fori_loop(0, NCH // 2, pair_body, 0)
    for par in range(2):
        pltpu.make_async_copy(mv[par], out_hbm.at[pl.ds(0, CH)],
                              semo[par]).wait()


def _sc_call(bv_flat, idx_3d, interpret=False):
    mesh = plsc.VectorSubcoreMesh(core_axis_name="c", subcore_axis_name="s")
    fn = functools.partial(
        pl.kernel,
        out_type=jax.ShapeDtypeStruct((BN_TOT, C2), jnp.float32),
        mesh=mesh,
        scratch_types=[
            pltpu.VMEM((NCH, CH * K), jnp.int32),
            pltpu.VMEM((CH * K, CP), jnp.float32),
            pltpu.VMEM((CH * K, CP), jnp.float32),
            pltpu.VMEM((CH, C2), jnp.float32),
            pltpu.VMEM((CH, C2), jnp.float32),
            pltpu.SemaphoreType.DMA,
            pltpu.SemaphoreType.DMA,
            pltpu.SemaphoreType.DMA,
            pltpu.SemaphoreType.DMA,
        ],
        interpret=interpret,
    )(_sc_body)
    return fn(bv_flat, idx_3d)


def _tc2_body(a_ref, m_ref, w2e_ref, b2e_ref, x_ref, o_ref):
    g = jnp.maximum(a_ref[0] + m_ref[0], 0.0)
    o = lax.dot_general(w2e_ref[...], g, (((1,), (1,)), ((), ())),
                        preferred_element_type=jnp.float32)
    o_ref[0] = o + b2e_ref[...] + x_ref[0]


def _tc2_call(apre, m, w2e, b2e, x3, interpret=False):
    return pl.pallas_call(
        _tc2_body,
        grid=(B,),
        in_specs=[
            pl.BlockSpec((1, N, C2), lambda b: (b, 0, 0)),
            pl.BlockSpec((1, N, C2), lambda b: (b, 0, 0)),
            pl.BlockSpec((C, C2), lambda b: (0, 0)),
            pl.BlockSpec((C, 1), lambda b: (0, 0)),
            pl.BlockSpec((1, C, N), lambda b: (b, 0, 0)),
        ],
        out_specs=pl.BlockSpec((1, C, N), lambda b: (b, 0, 0)),
        out_shape=jax.ShapeDtypeStruct((B, C, N), jnp.float32),
        interpret=interpret,
    )(apre, m, w2e, b2e, x3)


def _run(x, fc1_w, fc1_b, bn1_g, bn1_b, gc_w, gc_b, fc2_w, fc2_b, bn2_g, bn2_b,
         interpret=False, sc_interpret=False):
    x3 = x.reshape(B, C, N)
    s1 = bn1_g * lax.rsqrt(jnp.float32(1.0 + BN_EPS))
    w1e = fc1_w * s1[:, None]
    b1e = (fc1_b * s1 + bn1_b)[:, None]
    w1 = gc_w[:, :C]
    w2 = gc_w[:, C:]
    wat = (w1 - w2).T
    w2t = w2.T
    gcb = gc_b[None, :]
    s2 = bn2_g * lax.rsqrt(jnp.float32(1.0 + BN_EPS))
    w2e = fc2_w * s2[:, None]
    b2e = (fc2_b * s2 + bn2_b)[:, None]

    apre, bv, idx = _tc1_call(x3, w1e, b1e, wat, w2t, gcb, interpret=interpret)
    bv_flat = bv.reshape(BN_TOT, CP)
    idx_3d = idx.reshape(NW, NCH, CH * K)
    m = _sc_call(bv_flat, idx_3d, interpret=sc_interpret)
    m3 = m.reshape(B, N, C2)
    out = _tc2_call(apre, m3, w2e, b2e, x3, interpret=interpret)
    return out.reshape(B, C, H, W)


def kernel(x, fc1_w, fc1_b, bn1_g, bn1_b, gc_w, gc_b, fc2_w, fc2_b, bn2_g, bn2_b):
    return _run(x, fc1_w, fc1_b, bn1_g, bn1_b, gc_w, gc_b,
                fc2_w, fc2_b, bn2_g, bn2_b)
